# baseline (speedup 1.0000x reference)
import functools

import jax
import jax.numpy as jnp
from jax import lax
from jax.experimental import pallas as pl
from jax.experimental.pallas import tpu as pltpu
from jax.experimental.pallas import tpu_sc as plsc

NC = 2
NS = 16
NW = NC * NS
LANES = 16
CH = 128


def _bcast_lane(v, lane):
    idx = jnp.full((LANES, 1), lane, jnp.int32)
    dnums = lax.GatherDimensionNumbers(
        offset_dims=(), collapsed_slice_dims=(0,), start_index_map=(0,))
    return lax.gather(v, idx, dnums, (1,),
                      mode=lax.GatherScatterMode.PROMISE_IN_BOUNDS)


def _sc_aggregate(h, sd, ecp, nchunks):
    N, D = h.shape
    tpw = ecp.shape[1]
    npad = ((N + NS * CH - 1) // (NS * CH)) * NS * CH
    rows_per_tile = npad // NS
    nsub = rows_per_tile // CH
    sub = CH

    mesh = plsc.VectorSubcoreMesh(core_axis_name="c", subcore_axis_name="s")

    @functools.partial(
        pl.kernel,
        out_type=jax.ShapeDtypeStruct((NC, npad, D), jnp.float32),
        mesh=mesh,
        scratch_types=dict(
            idx0=pltpu.VMEM((2, CH), jnp.int32),
            ec_all=pltpu.VMEM((tpw, CH), jnp.float32),
            rows=pltpu.VMEM((CH, D), jnp.float32),
            acc=pltpu.VMEM_SHARED((npad, D), jnp.float32),
        ),
    )
    def agg_kernel(h_hbm, sd_hbm, ec_hbm, out_hbm, idx0, ec_all, rows, acc):
        cid = lax.axis_index("c")
        sid = lax.axis_index("s")
        wid = sid * NC + cid

        zero16 = jnp.zeros((LANES,), jnp.float32)

        def zrow(r, _):
            for j in range(D // LANES):
                rows[r, pl.ds(j * LANES, LANES)] = zero16
            return 0

        lax.fori_loop(0, sub, zrow, 0)
        for i in range(nsub):
            pltpu.sync_copy(rows,
                            acc.at[pl.ds(sid * rows_per_tile + i * sub, sub)])
        pltpu.sync_copy(ec_hbm.at[wid], ec_all)
        plsc.subcore_barrier()

        def chunk(t, _):
            pltpu.sync_copy(sd_hbm.at[wid + NW * t], idx0)
            pltpu.sync_copy(h_hbm.at[idx0.at[0]], rows)

            def group(gidx, _):
                ecg = ec_all[t, pl.ds(gidx * LANES, LANES)]
                for e in range(LANES):
                    w = _bcast_lane(ecg, e)
                    row = gidx * LANES + e
                    for j in range(D // LANES):
                        sl = pl.ds(j * LANES, LANES)
                        rows[row, sl] = rows[row, sl] * w
                return 0

            lax.fori_loop(0, CH // LANES, group, 0)
            pltpu.sync_copy(rows, acc.at[idx0.at[1]], add=True)
            return 0

        lax.fori_loop(0, tpw, chunk, 0)
        plsc.subcore_barrier()

        for i in range(nsub):
            r0 = sid * rows_per_tile + i * sub
            pltpu.sync_copy(acc.at[pl.ds(r0, sub)], rows)
            pltpu.sync_copy(rows, out_hbm.at[cid, pl.ds(r0, sub)])

    return agg_kernel(h, sd, ecp)


def _mlp_body(part_ref, h_ref, nc_ref, w1_ref, b1_ref, g_ref, be_ref, w2_ref,
              b2_ref, o_ref):
    n = h_ref.shape[0]
    agg = part_ref[0] + part_ref[1]
    xx = agg[:n] * nc_ref[...] + h_ref[...]
    h1 = jnp.dot(xx, w1_ref[...], preferred_element_type=jnp.float32)
    h1 = h1 + b1_ref[...]
    mu = jnp.mean(h1, axis=0, keepdims=True)
    var = jnp.mean((h1 - mu) ** 2, axis=0, keepdims=True)
    hn = (h1 - mu) / jnp.sqrt(var + 1e-5) * g_ref[...] + be_ref[...]
    hr = jnp.maximum(hn, 0.0)
    h2 = jnp.dot(hr, w2_ref[...], preferred_element_type=jnp.float32)
    o_ref[...] = jnp.maximum(h2 + b2_ref[...], 0.0)


def _tc_layer(part, h, nc, w1, b1, g, be, w2, b2):
    N, _ = h.shape
    return pl.pallas_call(
        _mlp_body,
        out_shape=jax.ShapeDtypeStruct((N, w2.shape[1]), jnp.float32),
    )(part, h, nc, w1, b1.reshape(1, -1), g.reshape(1, -1),
      be.reshape(1, -1), w2, b2.reshape(1, -1))


def _pool_body(h_ref, batch_ref, wc_ref, bc_ref, o_ref, *, nb):
    h = h_ref[...]
    seg = batch_ref[...]
    ids = lax.broadcasted_iota(jnp.int32, (nb, seg.shape[1]), 0)
    m = (ids == seg).astype(jnp.float32)
    cnt = jnp.sum(m, axis=1, keepdims=True)
    summed = jnp.dot(m, h, preferred_element_type=jnp.float32)
    pooled = summed / jnp.maximum(cnt, 1.0)
    o_ref[...] = jnp.dot(pooled, wc_ref[...],
                         preferred_element_type=jnp.float32) + bc_ref[...]


def _tc_pool(h, batch_row, wc, bc, nb):
    return pl.pallas_call(
        functools.partial(_pool_body, nb=nb),
        out_shape=jax.ShapeDtypeStruct((nb, wc.shape[1]), jnp.float32),
    )(h, batch_row, wc, bc.reshape(1, -1))


def kernel(x, edge_index, batch, node_centrality, edge_centrality,
           W1_0, b1_0, g_0, be_0, W2_0, b2_0,
           W1_1, b1_1, g_1, be_1, W2_1, b2_1,
           W1_2, b1_2, g_2, be_2, W2_2, b2_2,
           Wc, bc):
    src = edge_index[0]
    dst = edge_index[1]
    E = edge_index.shape[1]
    nchunks = -(-E // CH)
    tpw = (nchunks + NW - 1) // NW
    tpw = tpw + (tpw % 2)
    ncp = tpw * NW
    zpad = jnp.zeros((ncp * CH - E,), jnp.int32)
    sd = jnp.stack(
        [jnp.concatenate([src.astype(jnp.int32), zpad]).reshape(ncp, CH),
         jnp.concatenate([dst.astype(jnp.int32), zpad]).reshape(ncp, CH)],
        axis=1)
    ecpad = jnp.concatenate(
        [edge_centrality.astype(jnp.float32),
         jnp.zeros((ncp * CH - E,), jnp.float32)])
    ecp = ecpad.reshape(tpw, NW, CH).transpose(1, 0, 2)
    nc = node_centrality.reshape(-1, 1)
    batch_row = batch.reshape(1, -1).astype(jnp.int32)
    layers = [
        (W1_0, b1_0, g_0, be_0, W2_0, b2_0),
        (W1_1, b1_1, g_1, be_1, W2_1, b2_1),
        (W1_2, b1_2, g_2, be_2, W2_2, b2_2),
    ]
    h = x
    for (w1, b1, g, be, w2, b2) in layers:
        part = _sc_aggregate(h, sd, ecp, nchunks)
        h = _tc_layer(part, h, nc, w1, b1, g, be, w2, b2)
    return _tc_pool(h, batch_row, Wc, bc, 64)


# speedup vs baseline: 1.9470x; 1.9470x over previous
import functools

import jax
import jax.numpy as jnp
from jax import lax
from jax.experimental import pallas as pl
from jax.experimental.pallas import tpu as pltpu
from jax.experimental.pallas import tpu_sc as plsc

NC = 2
NS = 16
NW = NC * NS
LANES = 16
CH = 128


def _bcast_lane(v, lane):
    idx = jnp.full((LANES, 1), lane, jnp.int32)
    dnums = lax.GatherDimensionNumbers(
        offset_dims=(), collapsed_slice_dims=(0,), start_index_map=(0,))
    return lax.gather(v, idx, dnums, (1,),
                      mode=lax.GatherScatterMode.PROMISE_IN_BOUNDS)


def _sc_aggregate(h, sd, ecp, nchunks):
    N, D = h.shape
    tpw = ecp.shape[1]
    npad = ((N + NS * CH - 1) // (NS * CH)) * NS * CH
    rows_per_tile = npad // NS
    nsub = rows_per_tile // CH
    sub = CH

    mesh = plsc.VectorSubcoreMesh(core_axis_name="c", subcore_axis_name="s")

    @functools.partial(
        pl.kernel,
        out_type=jax.ShapeDtypeStruct((NC, npad, D), jnp.float32),
        mesh=mesh,
        scratch_types=dict(
            idx0=pltpu.VMEM((2, CH), jnp.int32),
            ec_all=pltpu.VMEM((tpw, CH), jnp.float32),
            rows=pltpu.VMEM((CH, D), jnp.float32),
            acc=pltpu.VMEM_SHARED((npad, D), jnp.float32),
        ),
    )
    def agg_kernel(h_hbm, sd_hbm, ec_hbm, out_hbm, idx0, ec_all, rows, acc):
        cid = lax.axis_index("c")
        sid = lax.axis_index("s")
        wid = sid * NC + cid

        zero16 = jnp.zeros((LANES,), jnp.float32)

        def zrow(r, _):
            for j in range(D // LANES):
                rows[r, pl.ds(j * LANES, LANES)] = zero16
            return 0

        lax.fori_loop(0, sub, zrow, 0)
        for i in range(nsub):
            pltpu.sync_copy(rows,
                            acc.at[pl.ds(sid * rows_per_tile + i * sub, sub)])
        pltpu.sync_copy(ec_hbm.at[wid], ec_all)
        plsc.subcore_barrier()

        def chunk(t, _):
            pltpu.sync_copy(sd_hbm.at[wid + NW * t], idx0)
            pltpu.sync_copy(h_hbm.at[idx0.at[0]], rows)

            def group(gidx, _):
                ecg = ec_all[t, pl.ds(gidx * LANES, LANES)]
                for e in range(LANES):
                    w = _bcast_lane(ecg, e)
                    row = gidx * LANES + e
                    for j in range(D // LANES):
                        sl = pl.ds(j * LANES, LANES)
                        rows[row, sl] = rows[row, sl] * w
                return 0

            lax.fori_loop(0, CH // LANES, group, 0)
            pltpu.sync_copy(rows, acc.at[idx0.at[1]], add=True)
            return 0

        lax.fori_loop(0, tpw, chunk, 0)
        plsc.subcore_barrier()

        for i in range(nsub):
            r0 = sid * rows_per_tile + i * sub
            pltpu.sync_copy(acc.at[pl.ds(r0, sub)], rows)
            pltpu.sync_copy(rows, out_hbm.at[cid, pl.ds(r0, sub)])

    return agg_kernel(h, sd, ecp)


def _mlp_body(part_ref, h_ref, nc_ref, w1_ref, b1_ref, g_ref, be_ref, w2_ref,
              b2_ref, o_ref):
    n = h_ref.shape[0]
    agg = part_ref[0] + part_ref[1]
    xx = agg[:n] * nc_ref[...] + h_ref[...]
    h1 = jnp.dot(xx, w1_ref[...], preferred_element_type=jnp.float32)
    h1 = h1 + b1_ref[...]
    mu = jnp.mean(h1, axis=0, keepdims=True)
    var = jnp.mean((h1 - mu) ** 2, axis=0, keepdims=True)
    hn = (h1 - mu) / jnp.sqrt(var + 1e-5) * g_ref[...] + be_ref[...]
    hr = jnp.maximum(hn, 0.0)
    h2 = jnp.dot(hr, w2_ref[...], preferred_element_type=jnp.float32)
    o_ref[...] = jnp.maximum(h2 + b2_ref[...], 0.0)


def _tc_layer(part, h, nc, w1, b1, g, be, w2, b2):
    N, _ = h.shape
    return pl.pallas_call(
        _mlp_body,
        out_shape=jax.ShapeDtypeStruct((N, w2.shape[1]), jnp.float32),
    )(part, h, nc, w1, b1.reshape(1, -1), g.reshape(1, -1),
      be.reshape(1, -1), w2, b2.reshape(1, -1))


def _pool_body(h_ref, batch_ref, wc_ref, bc_ref, o_ref, *, nb):
    h = h_ref[...]
    seg = batch_ref[...]
    ids = lax.broadcasted_iota(jnp.int32, (nb, seg.shape[1]), 0)
    m = (ids == seg).astype(jnp.float32)
    cnt = jnp.sum(m, axis=1, keepdims=True)
    summed = jnp.dot(m, h, preferred_element_type=jnp.float32)
    pooled = summed / jnp.maximum(cnt, 1.0)
    o_ref[...] = jnp.dot(pooled, wc_ref[...],
                         preferred_element_type=jnp.float32) + bc_ref[...]


def _tc_pool(h, batch_row, wc, bc, nb):
    return pl.pallas_call(
        functools.partial(_pool_body, nb=nb),
        out_shape=jax.ShapeDtypeStruct((nb, wc.shape[1]), jnp.float32),
    )(h, batch_row, wc, bc.reshape(1, -1))


def kernel(x, edge_index, batch, node_centrality, edge_centrality,
           W1_0, b1_0, g_0, be_0, W2_0, b2_0,
           W1_1, b1_1, g_1, be_1, W2_1, b2_1,
           W1_2, b1_2, g_2, be_2, W2_2, b2_2,
           Wc, bc):
    src = edge_index[0]
    dst = edge_index[1]
    E = edge_index.shape[1]
    N = x.shape[0]
    nchunks = -(-E // CH)
    tpw = (nchunks + NW - 1) // NW
    tpw = tpw + (tpw % 2)
    ncp = tpw * NW
    spread = jnp.arange(ncp * CH - E, dtype=jnp.int32) % N
    sd = jnp.stack(
        [jnp.concatenate([src.astype(jnp.int32), spread]).reshape(ncp, CH),
         jnp.concatenate([dst.astype(jnp.int32), spread]).reshape(ncp, CH)],
        axis=1)
    ecpad = jnp.concatenate(
        [edge_centrality.astype(jnp.float32),
         jnp.zeros((ncp * CH - E,), jnp.float32)])
    ecp = ecpad.reshape(tpw, NW, CH).transpose(1, 0, 2)
    nc = node_centrality.reshape(-1, 1)
    batch_row = batch.reshape(1, -1).astype(jnp.int32)
    layers = [
        (W1_0, b1_0, g_0, be_0, W2_0, b2_0),
        (W1_1, b1_1, g_1, be_1, W2_1, b2_1),
        (W1_2, b1_2, g_2, be_2, W2_2, b2_2),
    ]
    h = x
    for (w1, b1, g, be, w2, b2) in layers:
        part = _sc_aggregate(h, sd, ecp, nchunks)
        h = _tc_layer(part, h, nc, w1, b1, g, be, w2, b2)
    return _tc_pool(h, batch_row, Wc, bc, 64)


# speedup vs baseline: 3.0255x; 1.5539x over previous
import functools

import jax
import jax.numpy as jnp
from jax import lax
from jax.experimental import pallas as pl
from jax.experimental.pallas import tpu as pltpu
from jax.experimental.pallas import tpu_sc as plsc

NC = 2
NS = 16
NW = NC * NS
LANES = 16
CH = 128


def _bcast_lane(v, lane):
    idx = jnp.full((LANES, 1), lane, jnp.int32)
    dnums = lax.GatherDimensionNumbers(
        offset_dims=(), collapsed_slice_dims=(0,), start_index_map=(0,))
    return lax.gather(v, idx, dnums, (1,),
                      mode=lax.GatherScatterMode.PROMISE_IN_BOUNDS)


def _sc_aggregate(h, sd, ecp, nchunks):
    N, D = h.shape
    tpw = ecp.shape[1]
    npad = ((N + NS * CH - 1) // (NS * CH)) * NS * CH
    rows_per_tile = npad // NS
    nsub = rows_per_tile // CH
    sub = CH

    mesh = plsc.VectorSubcoreMesh(core_axis_name="c", subcore_axis_name="s")

    @functools.partial(
        pl.kernel,
        out_type=jax.ShapeDtypeStruct((NC, npad, D), jnp.float32),
        mesh=mesh,
        scratch_types=dict(
            idx0=pltpu.VMEM((2, CH), jnp.int32),
            idx1=pltpu.VMEM((2, CH), jnp.int32),
            ec_all=pltpu.VMEM((tpw, CH), jnp.float32),
            rows0=pltpu.VMEM((CH, D), jnp.float32),
            rows1=pltpu.VMEM((CH, D), jnp.float32),
            acc=pltpu.VMEM_SHARED((npad, D), jnp.float32),
            sem_g0=pltpu.SemaphoreType.DMA,
            sem_g1=pltpu.SemaphoreType.DMA,
        ),
    )
    def agg_kernel(h_hbm, sd_hbm, ec_hbm, out_hbm, idx0, idx1, ec_all,
                   rows0, rows1, acc, sem_g0, sem_g1):
        cid = lax.axis_index("c")
        sid = lax.axis_index("s")
        wid = sid * NC + cid

        zero16 = jnp.zeros((LANES,), jnp.float32)

        def zrow(r, _):
            for j in range(D // LANES):
                rows0[r, pl.ds(j * LANES, LANES)] = zero16
            return 0

        lax.fori_loop(0, sub, zrow, 0)
        for i in range(nsub):
            pltpu.sync_copy(rows0,
                            acc.at[pl.ds(sid * rows_per_tile + i * sub, sub)])
        pltpu.sync_copy(ec_hbm.at[wid], ec_all)
        plsc.subcore_barrier()

        idx = (idx0, idx1)
        rws = (rows0, rows1)
        sem_g = (sem_g0, sem_g1)

        def gather(b):
            return pltpu.make_async_copy(
                h_hbm.at[idx[b].at[0]], rws[b], sem_g[b])

        pltpu.sync_copy(sd_hbm.at[wid], idx0)
        gather(0).start()

        def pair(t2, _):
            for b in (0, 1):
                t = 2 * t2 + b
                nb = 1 - b

                @pl.when(t + 1 < tpw)
                def _():
                    pltpu.sync_copy(sd_hbm.at[wid + NW * (t + 1)], idx[nb])
                    gather(nb).start()

                gather(b).wait()

                def group(gidx, _):
                    ecg = ec_all[t, pl.ds(gidx * LANES, LANES)]
                    for e in range(LANES):
                        w = _bcast_lane(ecg, e)
                        row = gidx * LANES + e
                        for j in range(D // LANES):
                            sl = pl.ds(j * LANES, LANES)
                            rws[b][row, sl] = rws[b][row, sl] * w
                    return 0

                lax.fori_loop(0, CH // LANES, group, 0)
                pltpu.sync_copy(rws[b], acc.at[idx[b].at[1]], add=True)
            return 0

        lax.fori_loop(0, tpw // 2, pair, 0)
        plsc.subcore_barrier()

        for i in range(nsub):
            r0 = sid * rows_per_tile + i * sub
            pltpu.sync_copy(acc.at[pl.ds(r0, sub)], rows0)
            pltpu.sync_copy(rows0, out_hbm.at[cid, pl.ds(r0, sub)])

    return agg_kernel(h, sd, ecp)


def _mlp_body(part_ref, h_ref, nc_ref, w1_ref, b1_ref, g_ref, be_ref, w2_ref,
              b2_ref, o_ref):
    n = h_ref.shape[0]
    agg = part_ref[0] + part_ref[1]
    xx = agg[:n] * nc_ref[...] + h_ref[...]
    h1 = jnp.dot(xx, w1_ref[...], preferred_element_type=jnp.float32)
    h1 = h1 + b1_ref[...]
    mu = jnp.mean(h1, axis=0, keepdims=True)
    var = jnp.mean((h1 - mu) ** 2, axis=0, keepdims=True)
    hn = (h1 - mu) / jnp.sqrt(var + 1e-5) * g_ref[...] + be_ref[...]
    hr = jnp.maximum(hn, 0.0)
    h2 = jnp.dot(hr, w2_ref[...], preferred_element_type=jnp.float32)
    o_ref[...] = jnp.maximum(h2 + b2_ref[...], 0.0)


def _tc_layer(part, h, nc, w1, b1, g, be, w2, b2):
    N, _ = h.shape
    return pl.pallas_call(
        _mlp_body,
        out_shape=jax.ShapeDtypeStruct((N, w2.shape[1]), jnp.float32),
    )(part, h, nc, w1, b1.reshape(1, -1), g.reshape(1, -1),
      be.reshape(1, -1), w2, b2.reshape(1, -1))


def _pool_body(h_ref, batch_ref, wc_ref, bc_ref, o_ref, *, nb):
    h = h_ref[...]
    seg = batch_ref[...]
    ids = lax.broadcasted_iota(jnp.int32, (nb, seg.shape[1]), 0)
    m = (ids == seg).astype(jnp.float32)
    cnt = jnp.sum(m, axis=1, keepdims=True)
    summed = jnp.dot(m, h, preferred_element_type=jnp.float32)
    pooled = summed / jnp.maximum(cnt, 1.0)
    o_ref[...] = jnp.dot(pooled, wc_ref[...],
                         preferred_element_type=jnp.float32) + bc_ref[...]


def _tc_pool(h, batch_row, wc, bc, nb):
    return pl.pallas_call(
        functools.partial(_pool_body, nb=nb),
        out_shape=jax.ShapeDtypeStruct((nb, wc.shape[1]), jnp.float32),
    )(h, batch_row, wc, bc.reshape(1, -1))


def kernel(x, edge_index, batch, node_centrality, edge_centrality,
           W1_0, b1_0, g_0, be_0, W2_0, b2_0,
           W1_1, b1_1, g_1, be_1, W2_1, b2_1,
           W1_2, b1_2, g_2, be_2, W2_2, b2_2,
           Wc, bc):
    src = edge_index[0]
    dst = edge_index[1]
    E = edge_index.shape[1]
    N = x.shape[0]
    nchunks = -(-E // CH)
    tpw = (nchunks + NW - 1) // NW
    tpw = tpw + (tpw % 2)
    ncp = tpw * NW
    spread = jnp.arange(ncp * CH - E, dtype=jnp.int32) % N
    sd = jnp.stack(
        [jnp.concatenate([src.astype(jnp.int32), spread]).reshape(ncp, CH),
         jnp.concatenate([dst.astype(jnp.int32), spread]).reshape(ncp, CH)],
        axis=1)
    ecpad = jnp.concatenate(
        [edge_centrality.astype(jnp.float32),
         jnp.zeros((ncp * CH - E,), jnp.float32)])
    ecp = ecpad.reshape(tpw, NW, CH).transpose(1, 0, 2)
    nc = node_centrality.reshape(-1, 1)
    batch_row = batch.reshape(1, -1).astype(jnp.int32)
    layers = [
        (W1_0, b1_0, g_0, be_0, W2_0, b2_0),
        (W1_1, b1_1, g_1, be_1, W2_1, b2_1),
        (W1_2, b1_2, g_2, be_2, W2_2, b2_2),
    ]
    h = x
    for (w1, b1, g, be, w2, b2) in layers:
        part = _sc_aggregate(h, sd, ecp, nchunks)
        h = _tc_layer(part, h, nc, w1, b1, g, be, w2, b2)
    return _tc_pool(h, batch_row, Wc, bc, 64)
